# baseline (speedup 1.0000x reference)
import functools

import jax
import jax.numpy as jnp
from jax import lax
from jax.experimental import pallas as pl
from jax.experimental.pallas import tpu as pltpu
from jax.experimental.pallas import tpu_sc as plsc

_NUM_HEADS = 12
_KEEP_RATIO = 0.7
_PP = 408
_IP = 416
_CH = 24


def _bf(a):
    return a.astype(jnp.bfloat16)


def _stage_a_body(x_ref, wq_ref, bq_ref, qkv_ref, kidx_ref, nsc_ref,
                  fidx_ref, *, N, C, H, keep):
    D = C // H
    NP = keep + 1
    scale = D ** -0.5
    f32 = jnp.float32

    qkv = jnp.dot(x_ref[0], wq_ref[...],
                  preferred_element_type=f32) + bq_ref[...]
    qkv_ref[0] = qkv
    qkvb = _bf(qkv)

    k_part = qkvb[:, C:2 * C]
    q_cls = qkvb[0:1, 0:C].astype(f32) * scale
    ic_r = lax.broadcasted_iota(jnp.int32, (C, C), 0)
    ic_c = lax.broadcasted_iota(jnp.int32, (C, C), 1)
    q_col = jnp.sum(jnp.where(ic_r == ic_c, q_cls, 0.0),
                    axis=1, keepdims=True)
    HP = 128
    ih_r = lax.broadcasted_iota(jnp.int32, (C, HP), 0)
    ih_c = lax.broadcasted_iota(jnp.int32, (C, HP), 1)
    m_sel = jnp.where(ih_c == ih_r // D, q_col, 0.0)
    logits = jnp.dot(k_part, _bf(m_sel), preferred_element_type=f32)
    lmax = jnp.max(logits, axis=0, keepdims=True)
    lexp = jnp.exp(logits - lmax)
    lsum = jnp.sum(lexp, axis=0, keepdims=True)
    probs = lexp / lsum
    head_ok = lax.broadcasted_iota(jnp.int32, (N, HP), 1) < H
    s_col = jnp.sum(jnp.where(head_ok, probs, 0.0),
                    axis=1, keepdims=True) / H

    in_r = lax.broadcasted_iota(jnp.int32, (N, N), 0)
    in_c = lax.broadcasted_iota(jnp.int32, (N, N), 1)
    s_row = jnp.sum(jnp.where(in_r == in_c, s_col, 0.0),
                    axis=0, keepdims=True)
    prefer = ((in_r >= 1) & (in_c >= 1)
              & ((s_col > s_row) | ((s_col == s_row) & (in_r < in_c))))
    rank_row = jnp.sum(prefer.astype(f32), axis=0, keepdims=True)
    kept_row = rank_row < keep
    kept_f = kept_row.astype(f32)
    kept_col = jnp.sum(jnp.where(in_r == in_c, kept_f, 0.0),
                       axis=1, keepdims=True)
    pos_row = jnp.sum(kept_col * (in_r < in_c).astype(f32),
                      axis=0, keepdims=True)

    ip_p = lax.broadcasted_iota(jnp.int32, (NP, N), 0).astype(f32)
    oh = jnp.where(kept_row & (pos_row == ip_p), 1.0, 0.0)

    j_row = lax.broadcasted_iota(jnp.int32, (1, N), 1).astype(f32)
    kidx = jnp.sum(oh * j_row, axis=1, keepdims=True)
    nsc = jnp.sum(oh * s_row, axis=1, keepdims=True)
    kidx_ref[0] = kidx.astype(jnp.int32)
    nsc_ref[0] = nsc

    it_r = lax.broadcasted_iota(jnp.int32, (NP, _IP), 0)
    it_c = lax.broadcasted_iota(jnp.int32, (NP, _IP), 1)
    kidx_pad_row = jnp.sum(jnp.where(it_r == it_c, kidx, 0.0),
                           axis=0, keepdims=True)
    basef = (pl.program_id(0) * N).astype(f32)
    valid = (lax.broadcasted_iota(jnp.int32, (1, _IP), 1) < NP).astype(f32)
    fidx_ref[0] = (kidx_pad_row + basef * valid).astype(jnp.int32)


def _stage_a(x, W_qkv, b_qkv, N, C, H, keep):
    B = x.shape[0]
    C3 = W_qkv.shape[1]
    NP = keep + 1
    body = functools.partial(_stage_a_body, N=N, C=C, H=H, keep=keep)
    return pl.pallas_call(
        body,
        grid=(B,),
        in_specs=[
            pl.BlockSpec((1, N, C), lambda b: (b, 0, 0)),
            pl.BlockSpec((C, C3), lambda b: (0, 0)),
            pl.BlockSpec((1, C3), lambda b: (0, 0)),
        ],
        out_specs=[
            pl.BlockSpec((1, N, C3), lambda b: (b, 0, 0)),
            pl.BlockSpec((1, NP, 1), lambda b: (b, 0, 0)),
            pl.BlockSpec((1, NP, 1), lambda b: (b, 0, 0)),
            pl.BlockSpec((1, 1, _IP), lambda b: (b, 0, 0)),
        ],
        out_shape=[
            jax.ShapeDtypeStruct((B, N, C3), jnp.float32),
            jax.ShapeDtypeStruct((B, NP, 1), jnp.int32),
            jax.ShapeDtypeStruct((B, NP, 1), jnp.float32),
            jax.ShapeDtypeStruct((B, 1, _IP), jnp.int32),
        ],
    )(_bf(x), _bf(W_qkv), b_qkv.reshape(1, C3))


def _sc_gather(qkv2d, fidx, B, C3):
    mesh = plsc.VectorSubcoreMesh(core_axis_name="c", subcore_axis_name="s")
    nch = _PP // _CH

    @functools.partial(
        pl.kernel, mesh=mesh,
        out_type=jax.ShapeDtypeStruct((B * _PP, C3), jnp.float32),
        scratch_types=[
            pltpu.VMEM((_IP,), jnp.int32),
            pltpu.VMEM((_CH, C3), jnp.float32),
            pltpu.VMEM((_CH, C3), jnp.float32),
            pltpu.SemaphoreType.DMA,
            pltpu.SemaphoreType.DMA,
            pltpu.SemaphoreType.DMA,
            pltpu.SemaphoreType.DMA,
        ])
    def gather_kernel(qkv_hbm, fidx_hbm, out_hbm,
                      idx_v, buf0, buf1, g0, g1, w0, w1):
        wid = lax.axis_index("s") * 2 + lax.axis_index("c")
        pltpu.sync_copy(fidx_hbm.at[wid], idx_v)
        bufs = (buf0, buf1)
        gsems = (g0, g1)
        wsems = (w0, w1)
        whandles = [None, None]
        for i in range(nch):
            b = i % 2
            if whandles[b] is not None:
                whandles[b].wait()
            gh = pltpu.async_copy(
                qkv_hbm.at[idx_v.at[pl.ds(i * _CH, _CH)]], bufs[b], gsems[b])
            gh.wait()
            whandles[b] = pltpu.async_copy(
                bufs[b], out_hbm.at[pl.ds(wid * _PP + i * _CH, _CH)],
                wsems[b])
        for h in whandles:
            if h is not None:
                h.wait()

    return gather_kernel(qkv2d, fidx)


def _stage_c_body(g_ref, wp_ref, bp_ref, out_ref, *, C, H, NP):
    D = C // H
    scale = D ** -0.5
    f32 = jnp.float32
    gb = _bf(g_ref[0][:NP])
    onescol = (lax.broadcasted_iota(jnp.int32, (NP, D), 1) == 0)
    onescol = onescol.astype(jnp.bfloat16)
    outs = []
    for h in range(H):
        qh = gb[:, h * D:(h + 1) * D] * jnp.bfloat16(scale)
        kh = gb[:, C + h * D:C + (h + 1) * D]
        vh = gb[:, 2 * C + h * D:2 * C + (h + 1) * D]
        s_att = lax.dot_general(qh, kh, (((1,), (1,)), ((), ())),
                                preferred_element_type=f32)
        pb = _bf(jnp.exp(s_att))
        vaug = jnp.concatenate([vh, onescol], axis=1)
        o_aug = jnp.dot(pb, vaug, preferred_element_type=f32)
        rs = 1.0 / o_aug[:, D:D + 1]
        outs.append(o_aug[:, :D] * rs)
    att = jnp.concatenate(outs, axis=1)
    out_ref[0] = jnp.dot(_bf(att), wp_ref[...],
                         preferred_element_type=f32) + bp_ref[...]


def _stage_c(gpad, W_proj, b_proj, B, C, H, NP):
    C3 = 3 * C
    body = functools.partial(_stage_c_body, C=C, H=H, NP=NP)
    return pl.pallas_call(
        body,
        grid=(B,),
        in_specs=[
            pl.BlockSpec((1, _PP, C3), lambda b: (b, 0, 0)),
            pl.BlockSpec((C, C), lambda b: (0, 0)),
            pl.BlockSpec((1, C), lambda b: (0, 0)),
        ],
        out_specs=pl.BlockSpec((1, NP, C), lambda b: (b, 0, 0)),
        out_shape=jax.ShapeDtypeStruct((B, NP, C), jnp.float32),
    )(gpad, _bf(W_proj), b_proj.reshape(1, C))


def kernel(x, W_qkv, b_qkv, W_proj, b_proj):
    B, N, C = x.shape
    C3 = W_qkv.shape[1]
    H = _NUM_HEADS
    keep = max(1, int(_KEEP_RATIO * (N - 1)))
    NP = keep + 1

    qkv, kidx, nsc, fidx = _stage_a(x, W_qkv, b_qkv, N, C, H, keep)
    gflat = _sc_gather(qkv.reshape(B * N, C3), fidx.reshape(B, _IP), B, C3)
    gpad = gflat.reshape(B, _PP, C3)
    out = _stage_c(gpad, W_proj, b_proj, B, C, H, NP)
    return (out, kidx[..., 0], nsc[..., 0])
